# baseline (speedup 1.0000x reference)
import functools

import jax
import jax.numpy as jnp
from jax import lax
from jax.experimental import pallas as pl
from jax.experimental.pallas import tpu as pltpu
from jax.experimental.pallas import tpu_sc as plsc

_B = 4096
_S = 200
_D = 128
_NW = 32
_ROWS = _B * _S
_RPW = _ROWS // _NW
_C = 128
_NCH = _RPW // _C
_SE = 192 + _C
_LANES = 16
_NBUF = 3
_LOOK = 2
_NSTEADY = (_NCH // _NBUF) * _NBUF


def _sc_embed(ids2d, table, pos_ext):
    mesh = plsc.VectorSubcoreMesh(core_axis_name="c", subcore_axis_name="s")

    @functools.partial(
        pl.kernel,
        out_type=jax.ShapeDtypeStruct((_ROWS, _D), jnp.float32),
        mesh=mesh,
        scratch_types=[
            pltpu.VMEM((_NCH, _C), jnp.int32),
            pltpu.VMEM((_SE, _D), jnp.float32),
        ]
        + [pltpu.VMEM((_C, _D), jnp.float32) for _ in range(_NBUF)]
        + [pltpu.SemaphoreType.DMA for _ in range(2 * _NBUF)],
    )
    def k(ids_hbm, table_hbm, pos_hbm, out_hbm, idx_v, pos_v, *bufs_sems):
        rows = bufs_sems[:_NBUF]
        gsem = bufs_sems[_NBUF : 2 * _NBUF]
        osem = bufs_sems[2 * _NBUF :]

        cid = lax.axis_index("c")
        sid = lax.axis_index("s")
        wid = sid * 2 + cid
        base_row = wid * _RPW
        idx_base = wid * _NCH

        pltpu.sync_copy(pos_hbm, pos_v)
        pltpu.sync_copy(ids_hbm.at[pl.ds(idx_base, _NCH)], idx_v)

        def g_start(j, t):
            pltpu.async_copy(table_hbm.at[idx_v.at[j]], rows[t], gsem[t])

        def g_wait(t):
            pltpu.make_async_copy(
                table_hbm.at[idx_v.at[0]], rows[t], gsem[t]
            ).wait()

        def o_start(j, t):
            pltpu.async_copy(
                rows[t], out_hbm.at[pl.ds(base_row + j * _C, _C)], osem[t]
            )

        def o_wait(t):
            pltpu.make_async_copy(
                rows[t], out_hbm.at[pl.ds(base_row, _C)], osem[t]
            ).wait()

        def add_rows(t, off):
            def row_body(r, c2):
                rr = r * 2
                for u in range(2):
                    for cc in range(_D // _LANES):
                        sl = pl.ds(cc * _LANES, _LANES)
                        plsc.addupdate(
                            rows[t].at[rr + u, sl], pos_v[off + rr + u, sl]
                        )
                return c2

            lax.fori_loop(0, _C // 2, row_body, 0)

        for t in range(_LOOK):
            g_start(t, t)

        def body(i, carry):
            for t in range(_NBUF):
                j = i * _NBUF + t
                t2 = (t + _LOOK) % _NBUF
                g_wait(t)
                add_rows(t, lax.rem(j * _C, _S))
                o_start(j, t)

                @pl.when(j >= _NBUF - _LOOK)
                def _():
                    o_wait(t2)

                @pl.when(j + _LOOK < _NCH)
                def _():
                    g_start(j + _LOOK, t2)

            return carry

        lax.fori_loop(0, _NSTEADY // _NBUF, body, 0)
        for j in range(_NSTEADY, _NCH):
            t = j % _NBUF
            g_wait(t)
            add_rows(t, (j * _C) % _S)
            o_start(j, t)
            if j + (_NBUF - _LOOK) < _NCH:
                o_wait((t + _LOOK) % _NBUF)
        for j in range(_NCH - (_NBUF - _LOOK) - 1, _NCH):
            o_wait(j % _NBUF)

    return k(ids2d, table, pos_ext)


def kernel(input_ids, lin_embed_weight, pos_embed):
    ids2d = input_ids.reshape(_ROWS // _C, _C).astype(jnp.int32)
    pos2d = pos_embed.reshape(_S, _D)
    pos_ext = jnp.concatenate([pos2d, pos2d[: _SE - _S]], axis=0)
    out = _sc_embed(ids2d, lin_embed_weight, pos_ext)
    return out.reshape(_B, _S, _D)


# speedup vs baseline: 1.0281x; 1.0281x over previous
import functools

import jax
import jax.numpy as jnp
from jax import lax
from jax.experimental import pallas as pl
from jax.experimental.pallas import tpu as pltpu
from jax.experimental.pallas import tpu_sc as plsc

_B = 4096
_S = 200
_D = 128
_NW = 32
_ROWS = _B * _S
_RPW = _ROWS // _NW
_C = 80
_NCH = _RPW // _C
_SE = _S
_LANES = 16
_NBUF = 6
_LOOK = 4
_NSTEADY = (_NCH // _NBUF) * _NBUF


def _sc_embed(ids2d, table, pos_ext):
    mesh = plsc.VectorSubcoreMesh(core_axis_name="c", subcore_axis_name="s")

    @functools.partial(
        pl.kernel,
        out_type=jax.ShapeDtypeStruct((_ROWS, _D), jnp.float32),
        mesh=mesh,
        scratch_types=[
            pltpu.VMEM((_NCH, _C), jnp.int32),
            pltpu.VMEM((_SE, _D), jnp.float32),
        ]
        + [pltpu.VMEM((_C, _D), jnp.float32) for _ in range(_NBUF)]
        + [pltpu.SemaphoreType.DMA for _ in range(2 * _NBUF)],
    )
    def k(ids_hbm, table_hbm, pos_hbm, out_hbm, idx_v, pos_v, *bufs_sems):
        rows = bufs_sems[:_NBUF]
        gsem = bufs_sems[_NBUF : 2 * _NBUF]
        osem = bufs_sems[2 * _NBUF :]

        cid = lax.axis_index("c")
        sid = lax.axis_index("s")
        wid = sid * 2 + cid
        base_row = wid * _RPW
        idx_base = wid * _NCH

        pltpu.sync_copy(pos_hbm, pos_v)
        pltpu.sync_copy(ids_hbm.at[pl.ds(idx_base, _NCH)], idx_v)

        def g_start(j, t):
            pltpu.async_copy(table_hbm.at[idx_v.at[j]], rows[t], gsem[t])

        def g_wait(t):
            pltpu.make_async_copy(
                table_hbm.at[idx_v.at[0]], rows[t], gsem[t]
            ).wait()

        def o_start(j, t):
            pltpu.async_copy(
                rows[t], out_hbm.at[pl.ds(base_row + j * _C, _C)], osem[t]
            )

        def o_wait(t):
            pltpu.make_async_copy(
                rows[t], out_hbm.at[pl.ds(base_row, _C)], osem[t]
            ).wait()

        def add_rows(t, off):
            n1 = lax.min(_S - off, _C)

            def body1(r, c2):
                rr = r * 2
                for u in range(2):
                    for cc in range(_D // _LANES):
                        sl = pl.ds(cc * _LANES, _LANES)
                        plsc.addupdate(
                            rows[t].at[rr + u, sl], pos_v[off + rr + u, sl]
                        )
                return c2

            def body2(r, c2):
                rr = r * 2
                for u in range(2):
                    for cc in range(_D // _LANES):
                        sl = pl.ds(cc * _LANES, _LANES)
                        plsc.addupdate(
                            rows[t].at[n1 + rr + u, sl], pos_v[rr + u, sl]
                        )
                return c2

            lax.fori_loop(0, n1 // 2, body1, 0)
            lax.fori_loop(0, (_C - n1) // 2, body2, 0)

        for t in range(_LOOK):
            g_start(t, t)

        def body(i, carry):
            for t in range(_NBUF):
                j = i * _NBUF + t
                t2 = (t + _LOOK) % _NBUF
                g_wait(t)
                add_rows(t, lax.rem(j * _C, _S))
                o_start(j, t)

                @pl.when(j >= _NBUF - _LOOK)
                def _():
                    o_wait(t2)

                @pl.when(j + _LOOK < _NCH)
                def _():
                    g_start(j + _LOOK, t2)

            return carry

        lax.fori_loop(0, _NSTEADY // _NBUF, body, 0)
        for j in range(_NSTEADY, _NCH):
            t = j % _NBUF
            g_wait(t)
            add_rows(t, jnp.int32((j * _C) % _S))
            o_start(j, t)
            o_wait((t + _LOOK) % _NBUF)
        for j in range(_NCH - (_NBUF - _LOOK), _NCH):
            o_wait(j % _NBUF)

    return k(ids2d, table, pos_ext)


def kernel(input_ids, lin_embed_weight, pos_embed):
    ids2d = input_ids.reshape(_ROWS // _C, _C).astype(jnp.int32)
    pos2d = pos_embed.reshape(_S, _D)
    out = _sc_embed(ids2d, lin_embed_weight, pos2d)
    return out.reshape(_B, _S, _D)


# speedup vs baseline: 2.5887x; 2.5179x over previous
import functools

import jax
import jax.numpy as jnp
from jax import lax
from jax.experimental import pallas as pl
from jax.experimental.pallas import tpu as pltpu
from jax.experimental.pallas import tpu_sc as plsc

_B = 4096
_S = 200
_D = 128
_NW = 32
_ROWS = _B * _S
_RPW = _ROWS // _NW
_C = 80
_NCH = _RPW // _C
_LANES = 16
_NBUF = 5
_LOOK = 4

_OFFS = [(t * _C) % _S for t in range(_NBUF)]
_SE = max(_OFFS) + _C

assert _RPW % _C == 0 and _C % 8 == 0 and _C <= 128
assert (_NBUF * _C) % _S == 0 and _NCH % _NBUF == 0
assert 1 <= _NBUF - _LOOK


def _sc_embed(ids2d, table, pos_ext):
    mesh = plsc.VectorSubcoreMesh(core_axis_name="c", subcore_axis_name="s")

    @functools.partial(
        pl.kernel,
        out_type=jax.ShapeDtypeStruct((_ROWS, _D), jnp.float32),
        mesh=mesh,
        scratch_types=[
            pltpu.VMEM((_NCH, _C), jnp.int32),
            pltpu.VMEM((_SE, _D), jnp.float32),
        ]
        + [pltpu.VMEM((_C, _D), jnp.float32) for _ in range(_NBUF)]
        + [pltpu.SemaphoreType.DMA for _ in range(2 * _NBUF)],
    )
    def k(ids_hbm, table_hbm, pos_hbm, out_hbm, idx_v, pos_v, *bufs_sems):
        rows = bufs_sems[:_NBUF]
        gsem = bufs_sems[_NBUF : 2 * _NBUF]
        osem = bufs_sems[2 * _NBUF :]

        cid = lax.axis_index("c")
        sid = lax.axis_index("s")
        wid = sid * 2 + cid
        base_row = wid * _RPW
        idx_base = wid * _NCH

        pltpu.sync_copy(pos_hbm, pos_v)
        pltpu.sync_copy(ids_hbm.at[pl.ds(idx_base, _NCH)], idx_v)

        def g_start(j, t):
            pltpu.async_copy(table_hbm.at[idx_v.at[j]], rows[t], gsem[t])

        def g_wait(t):
            pltpu.make_async_copy(
                table_hbm.at[idx_v.at[0]], rows[t], gsem[t]
            ).wait()

        def o_start(j, t):
            pltpu.async_copy(
                rows[t], out_hbm.at[pl.ds(base_row + j * _C, _C)], osem[t]
            )

        def o_wait(t):
            pltpu.make_async_copy(
                rows[t], out_hbm.at[pl.ds(base_row, _C)], osem[t]
            ).wait()

        def add_rows(t, off):
            def row_body(r, c2):
                rr = r * 2
                for u in range(2):
                    for cc in range(_D // _LANES):
                        sl = pl.ds(cc * _LANES, _LANES)
                        plsc.addupdate(
                            rows[t].at[rr + u, sl], pos_v[off + rr + u, sl]
                        )
                return c2

            lax.fori_loop(0, _C // 2, row_body, 0)

        for t in range(_LOOK):
            g_start(t, t)

        def body(i, carry):
            for t in range(_NBUF):
                j = i * _NBUF + t
                t2 = (t + _LOOK) % _NBUF
                g_wait(t)
                add_rows(t, _OFFS[t])
                o_start(j, t)

                @pl.when(j >= _NBUF - _LOOK)
                def _():
                    o_wait(t2)

                @pl.when(j + _LOOK < _NCH)
                def _():
                    g_start(j + _LOOK, t2)

            return carry

        lax.fori_loop(0, _NCH // _NBUF, body, 0)
        for j in range(_NCH - (_NBUF - _LOOK), _NCH):
            o_wait(j % _NBUF)

    return k(ids2d, table, pos_ext)


def kernel(input_ids, lin_embed_weight, pos_embed):
    ids2d = input_ids.reshape(_ROWS // _C, _C).astype(jnp.int32)
    pos2d = pos_embed.reshape(_S, _D)
    pos_ext = jnp.concatenate([pos2d, pos2d[: _SE - _S]], axis=0)
    out = _sc_embed(ids2d, lin_embed_weight, pos_ext)
    return out.reshape(_B, _S, _D)


# speedup vs baseline: 2.6153x; 1.0103x over previous
import functools

import jax
import jax.numpy as jnp
from jax import lax
from jax.experimental import pallas as pl
from jax.experimental.pallas import tpu as pltpu
from jax.experimental.pallas import tpu_sc as plsc

_B = 4096
_S = 200
_D = 128
_NW = 32
_ROWS = _B * _S
_RPW = _ROWS // _NW
_C = 80
_NCH = _RPW // _C
_LANES = 16
_NBUF = 5
_LOOK = 3

_OFFS = [(t * _C) % _S for t in range(_NBUF)]
_SE = max(_OFFS) + _C

assert _RPW % _C == 0 and _C % 8 == 0 and _C <= 128
assert (_NBUF * _C) % _S == 0 and _NCH % _NBUF == 0
assert 2 <= _NBUF - _LOOK


def _sc_embed(ids2d, table, pos_ext):
    mesh = plsc.VectorSubcoreMesh(core_axis_name="c", subcore_axis_name="s")

    @functools.partial(
        pl.kernel,
        out_type=jax.ShapeDtypeStruct((_ROWS, _D), jnp.float32),
        mesh=mesh,
        scratch_types=[
            pltpu.VMEM((_NCH, _C), jnp.int32),
            pltpu.VMEM((_SE, _D), jnp.float32),
        ]
        + [pltpu.VMEM((_C, _D), jnp.float32) for _ in range(_NBUF)]
        + [pltpu.SemaphoreType.DMA for _ in range(2 * _NBUF)],
    )
    def k(ids_hbm, table_hbm, pos_hbm, out_hbm, idx_v, pos_v, *bufs_sems):
        rows = bufs_sems[:_NBUF]
        gsem = bufs_sems[_NBUF : 2 * _NBUF]
        osem = bufs_sems[2 * _NBUF :]

        cid = lax.axis_index("c")
        sid = lax.axis_index("s")
        wid = sid * 2 + cid
        base_row = wid * _RPW
        idx_base = wid * _NCH

        pltpu.sync_copy(pos_hbm, pos_v)
        pltpu.sync_copy(ids_hbm.at[pl.ds(idx_base, _NCH)], idx_v)

        def g_start(j, t):
            pltpu.async_copy(table_hbm.at[idx_v.at[j]], rows[t], gsem[t])

        def g_wait(t):
            pltpu.make_async_copy(
                table_hbm.at[idx_v.at[0]], rows[t], gsem[t]
            ).wait()

        def o_start(j, t):
            pltpu.async_copy(
                rows[t], out_hbm.at[pl.ds(base_row + j * _C, _C)], osem[t]
            )

        def o_wait(t):
            pltpu.make_async_copy(
                rows[t], out_hbm.at[pl.ds(base_row, _C)], osem[t]
            ).wait()

        def add_rows(t, off):
            def row_body(r, c2):
                rr = r * 2
                for u in range(2):
                    for cc in range(_D // _LANES):
                        sl = pl.ds(cc * _LANES, _LANES)
                        plsc.addupdate(
                            rows[t].at[rr + u, sl], pos_v[off + rr + u, sl]
                        )
                return c2

            lax.fori_loop(0, _C // 2, row_body, 0)

        for t in range(_LOOK):
            g_start(t, t)

        def body(i, carry):
            for t in range(_NBUF):
                j = i * _NBUF + t
                t2 = (t + _LOOK) % _NBUF
                g_wait(t)
                add_rows(t, _OFFS[t])
                o_start(j, t)

                @pl.when(j >= _NBUF - _LOOK)
                def _():
                    o_wait(t2)

                @pl.when(j + _LOOK < _NCH)
                def _():
                    g_start(j + _LOOK, t2)

            return carry

        lax.fori_loop(0, _NCH // _NBUF, body, 0)
        for j in range(_NCH - (_NBUF - _LOOK), _NCH):
            o_wait(j % _NBUF)

    return k(ids2d, table, pos_ext)


def kernel(input_ids, lin_embed_weight, pos_embed):
    ids2d = input_ids.reshape(_ROWS // _C, _C).astype(jnp.int32)
    pos2d = pos_embed.reshape(_S, _D)
    pos_ext = jnp.concatenate([pos2d, pos2d[: _SE - _S]], axis=0)
    out = _sc_embed(ids2d, lin_embed_weight, pos_ext)
    return out.reshape(_B, _S, _D)
